# Initial kernel scaffold; baseline (speedup 1.0000x reference)
import jax
import jax.numpy as jnp
from jax.experimental import pallas as pl


def kernel(z, slot_keys, slot_values, slot_mask, sal_Wr, sal_Wi, score_bias, novelty_scale, key_Wr, key_Wi, val_Wr, val_Wi, qry_Wr, qry_Wi, norm_gamma):
    raise NotImplementedError("write your pallas kernel here")


# speedup vs baseline: 2.1079x; 2.1079x over previous
import jax
import jax.numpy as jnp
import numpy as np
from jax.experimental import pallas as pl
from jax.experimental.pallas import tpu as pltpu

B, L, DIM, S, TOPK, THR = 4, 2048, 768, 32, 8, 0.5
D2 = 2 * DIM
LB = 512
NL = L // LB
BF = jnp.bfloat16
F32 = jnp.float32


def _k1_stats(z2_ref, uv_ref, ps_ref, am_ref, rms_ref):
    x = z2_ref[0]
    x16 = x.astype(BF).astype(F32)
    u16 = uv_ref[0:1, :].astype(BF).astype(F32)
    v16 = uv_ref[1:2, :].astype(BF).astype(F32)
    pr = jnp.sum(x16 * u16, axis=1, keepdims=True)
    pi = jnp.sum(x16 * v16, axis=1, keepdims=True)
    ps_ref[0] = jnp.sqrt(pr * pr + pi * pi)
    sq = x * x
    pair = sq + jnp.roll(sq, -1, axis=1)
    lane = jax.lax.broadcasted_iota(jnp.int32, (LB, D2), 1)
    mag = jnp.where(lane % 2 == 0, jnp.sqrt(pair), 0.0)
    am_ref[0] = jnp.sum(mag, axis=1, keepdims=True) * (1.0 / DIM)
    msq = jnp.sum(sq, axis=1, keepdims=True) * (1.0 / DIM)
    rms_ref[0] = jnp.sqrt(msq + 1e-8)


def _shift_right(x, k, lane):
    return jnp.where(lane >= k, jnp.roll(x, k, axis=1), 0.0)


def _shift_left(x, k, lane):
    return jnp.where(lane < L - k, jnp.roll(x, -k, axis=1), 0.0)


def _t_row(col_ref):
    ones11 = jnp.ones((1, 1), F32)
    return jax.lax.dot_general(ones11, col_ref[0], (((1,), (1,)), ((), ())),
                               preferred_element_type=F32,
                               precision=jax.lax.Precision.HIGHEST)


def _k2_segment(ps_ref, am_ref, bias_ref, nov_ref, sal_ref, seg_ref, w_ref):
    lane = jax.lax.broadcasted_iota(jnp.int32, (1, L), 1)
    am = _t_row(am_ref)
    local = am
    for k in (1, 2):
        local = local + _shift_right(am, k, lane) + _shift_left(am, k, lane)
    local = local * 0.2
    novelty = (am - local) * nov_ref[0, 0]
    sal = jax.nn.sigmoid(_t_row(ps_ref) + novelty + bias_ref[0, 0])
    sal_ref[0] = sal
    above = sal > THR
    abovef = above.astype(F32)
    prev = _shift_right(abovef, 1, lane)
    start = jnp.where(above, 1.0 - prev, 0.0)
    cum = start
    k = 1
    while k < L:
        cum = cum + _shift_right(cum, k, lane)
        k *= 2
    span = cum - 1.0
    seg_ref[0] = jnp.where(above, jnp.minimum(span, float(S)), float(S))
    w_ref[0] = jnp.where(above, sal, 0.0)


def _k3_pool(z2_ref, seg_ref, w_ref, ev_ref, em_ref, accz, accw):
    li = pl.program_id(1)

    @pl.when(li == 0)
    def _():
        accz[...] = jnp.zeros_like(accz)
        accw[...] = jnp.zeros_like(accw)

    x = z2_ref[0]
    seg = seg_ref[0]
    w = w_ref[0]
    sid = jax.lax.broadcasted_iota(jnp.int32, (S + 1, 1), 0).astype(F32)
    oht = (seg == sid).astype(F32)
    ohtw = oht * w
    accz[...] += jnp.dot(ohtw, x, preferred_element_type=F32,
                         precision=jax.lax.Precision.HIGHEST)
    den = jnp.sum(ohtw, axis=1, keepdims=True)
    cnt = jnp.sum(oht * (w > 0.0), axis=1, keepdims=True)
    accw[...] += jnp.concatenate([den, cnt], axis=1)

    @pl.when(li == NL - 1)
    def _():
        den_f = accw[:S, 0:1]
        ev_ref[0] = accz[:S, :] / jnp.maximum(den_f, 1e-8)
        em_ref[0] = (accw[:S, 1:2] > 0.0).astype(F32)


def _k4_project(ev_ref, em_ref, sk_ref, sv_ref, sm_ref, wk_ref, wv_ref,
                nk_ref, nv_ref, nm_ref):
    e16 = ev_ref[0].astype(BF)
    m = em_ref[0]
    ek = jnp.dot(e16, wk_ref[...], preferred_element_type=F32)
    ev = jnp.dot(e16, wv_ref[...], preferred_element_type=F32)
    nk = m * ek + (1.0 - m) * sk_ref[0]
    nv = m * ev + (1.0 - m) * sv_ref[0]
    nk_ref[0] = nk
    nv_ref[0] = nv
    ones11 = jnp.ones((1, 1), F32)
    m_row = jax.lax.dot_general(ones11, m, (((1,), (1,)), ((), ())),
                                preferred_element_type=F32,
                                precision=jax.lax.Precision.HIGHEST)
    nm_ref[0] = jnp.minimum(sm_ref[0] + m_row, 1.0)


def _k5_retrieve(z2_ref, rms_ref, g2_ref, wq_ref, nk_ref, nv_ref, nm_ref,
                 out_ref):
    x = z2_ref[0]
    zn = (x / rms_ref[0]) * g2_ref[...]
    q = jnp.dot(zn.astype(BF), wq_ref[...], preferred_element_type=F32)
    s = jax.lax.dot_general(q.astype(BF), nk_ref[0].astype(BF),
                            (((1,), (1,)), ((), ())),
                            preferred_element_type=F32) * (1.0 / np.sqrt(DIM))
    s = s + (nm_ref[0] - 1.0) * 1e9
    cur = s
    m1 = jnp.max(cur, axis=1, keepdims=True)
    mk = m1
    for _ in range(TOPK - 1):
        cur = jnp.where(cur >= mk, -1e30, cur)
        mk = jnp.max(cur, axis=1, keepdims=True)
    w = jnp.where(s >= mk, jnp.exp(s - m1), 0.0)
    w = w / jnp.sum(w, axis=1, keepdims=True)
    out_ref[0] = jnp.dot(w.astype(BF), nv_ref[0].astype(BF),
                         preferred_element_type=F32)


def _interleave_rows(a, b):
    d = a.shape[0]
    return jnp.stack([a, b], axis=1).reshape(2 * d, a.shape[1])


def _c2r(wr, wi):
    d = wr.shape[0]
    ro = _interleave_rows(wr, -wi)
    io = _interleave_rows(wi, wr)
    return jnp.stack([ro, io], axis=2).reshape(2 * d, 2 * d)


@jax.jit
def kernel(z, slot_keys, slot_values, slot_mask, sal_Wr, sal_Wi, score_bias,
           novelty_scale, key_Wr, key_Wi, val_Wr, val_Wi, qry_Wr, qry_Wi,
           norm_gamma):
    z2 = z.reshape(B, L, D2)
    sk2 = slot_keys.reshape(B, S, D2)
    sv2 = slot_values.reshape(B, S, D2)
    sm3 = slot_mask.reshape(B, 1, S)
    u = jnp.stack([sal_Wr[:, 0], -sal_Wi[:, 0]], axis=1).reshape(D2)
    v = jnp.stack([sal_Wi[:, 0], sal_Wr[:, 0]], axis=1).reshape(D2)
    uv = jnp.stack([u, v], axis=0)
    wk2 = _c2r(key_Wr, key_Wi).astype(BF)
    wv2 = _c2r(val_Wr, val_Wi).astype(BF)
    wq2 = _c2r(qry_Wr, qry_Wi).astype(BF)
    g2 = jnp.repeat(norm_gamma, 2).reshape(1, D2)
    bias = score_bias.reshape(1, 1).astype(F32)
    nov = novelty_scale.reshape(1, 1).astype(F32)

    zspec = pl.BlockSpec((1, LB, D2), lambda b, l: (b, l, 0))
    rowspec = pl.BlockSpec((1, 1, LB), lambda b, l: (b, 0, l))
    rmsspec = pl.BlockSpec((1, LB, 1), lambda b, l: (b, l, 0))

    ps, am, rms = pl.pallas_call(
        _k1_stats,
        grid=(B, NL),
        in_specs=[zspec, pl.BlockSpec((2, D2), lambda b, l: (0, 0))],
        out_specs=[rmsspec, rmsspec, rmsspec],
        out_shape=[jax.ShapeDtypeStruct((B, L, 1), F32)] * 3,
    )(z2, uv)

    fullrow = pl.BlockSpec((1, 1, L), lambda b: (b, 0, 0))
    fullcol = pl.BlockSpec((1, L, 1), lambda b: (b, 0, 0))
    scalar = pl.BlockSpec((1, 1), lambda b: (0, 0))
    sal3, seg, wgt = pl.pallas_call(
        _k2_segment,
        grid=(B,),
        in_specs=[fullcol, fullcol, scalar, scalar],
        out_specs=[fullrow, fullrow, fullrow],
        out_shape=[jax.ShapeDtypeStruct((B, 1, L), F32)] * 3,
    )(ps, am, bias, nov)

    events, emask = pl.pallas_call(
        _k3_pool,
        grid=(B, NL),
        in_specs=[zspec, rowspec, rowspec],
        out_specs=[pl.BlockSpec((1, S, D2), lambda b, l: (b, 0, 0)),
                   pl.BlockSpec((1, S, 1), lambda b, l: (b, 0, 0))],
        out_shape=[jax.ShapeDtypeStruct((B, S, D2), F32),
                   jax.ShapeDtypeStruct((B, S, 1), F32)],
        scratch_shapes=[pltpu.VMEM((S + 1, D2), F32),
                        pltpu.VMEM((S + 1, 2), F32)],
    )(z2, seg, wgt)

    slotspec = pl.BlockSpec((1, S, D2), lambda b: (b, 0, 0))
    wspec = pl.BlockSpec((D2, D2), lambda b: (0, 0))
    nk2, nv2, nm3 = pl.pallas_call(
        _k4_project,
        grid=(B,),
        in_specs=[slotspec, pl.BlockSpec((1, S, 1), lambda b: (b, 0, 0)),
                  slotspec, slotspec,
                  pl.BlockSpec((1, 1, S), lambda b: (b, 0, 0)),
                  wspec, wspec],
        out_specs=[slotspec, slotspec,
                   pl.BlockSpec((1, 1, S), lambda b: (b, 0, 0))],
        out_shape=[jax.ShapeDtypeStruct((B, S, D2), F32),
                   jax.ShapeDtypeStruct((B, S, D2), F32),
                   jax.ShapeDtypeStruct((B, 1, S), F32)],
    )(events, emask, sk2, sv2, sm3, wk2, wv2)

    retr2 = pl.pallas_call(
        _k5_retrieve,
        grid=(B, NL),
        in_specs=[zspec, rmsspec, pl.BlockSpec((1, D2), lambda b, l: (0, 0)),
                  pl.BlockSpec((D2, D2), lambda b, l: (0, 0)),
                  pl.BlockSpec((1, S, D2), lambda b, l: (b, 0, 0)),
                  pl.BlockSpec((1, S, D2), lambda b, l: (b, 0, 0)),
                  pl.BlockSpec((1, 1, S), lambda b, l: (b, 0, 0))],
        out_specs=zspec,
        out_shape=jax.ShapeDtypeStruct((B, L, D2), F32),
    )(z2, rms, g2, wq2, nk2, nv2, nm3)

    retrieved = retr2.reshape(B, L, DIM, 2)
    new_keys = nk2.reshape(B, S, DIM, 2)
    new_values = nv2.reshape(B, S, DIM, 2)
    return retrieved, new_keys, new_values, nm3.reshape(B, S), sal3.reshape(B, L)


# speedup vs baseline: 2.8524x; 1.3532x over previous
import jax
import jax.numpy as jnp
import numpy as np
from jax.experimental import pallas as pl
from jax.experimental.pallas import tpu as pltpu

B, L, DIM, S, TOPK, THR = 4, 2048, 768, 32, 8, 0.5
LB = 512
NL = L // LB
F32 = jnp.float32
_HI = jax.lax.Precision.HIGHEST


def _k1_stats(zr_ref, zi_ref, wr_ref, wi_ref, ps_ref, am_ref, rms_ref):
    xr = zr_ref[0]
    xi = zi_ref[0]
    pr = (jnp.dot(xr, wr_ref[...], preferred_element_type=F32)
          - jnp.dot(xi, wi_ref[...], preferred_element_type=F32))
    pi = (jnp.dot(xr, wi_ref[...], preferred_element_type=F32)
          + jnp.dot(xi, wr_ref[...], preferred_element_type=F32))
    ps_ref[0] = jnp.sqrt(pr * pr + pi * pi)
    msq = xr * xr + xi * xi
    mag = jnp.sqrt(msq)
    am_ref[0] = jnp.sum(mag, axis=1, keepdims=True) * (1.0 / DIM)
    rms_ref[0] = jnp.sqrt(jnp.sum(msq, axis=1, keepdims=True) * (1.0 / DIM)
                          + 1e-8)


def _shift_right(x, k, lane):
    return jnp.where(lane >= k, jnp.roll(x, k, axis=1), 0.0)


def _shift_left(x, k, lane):
    return jnp.where(lane < L - k, jnp.roll(x, -k, axis=1), 0.0)


def _t_row(col_ref):
    ones11 = jnp.ones((1, 1), F32)
    return jax.lax.dot_general(ones11, col_ref[0], (((1,), (1,)), ((), ())),
                               preferred_element_type=F32,
                               precision=_HI)


def _k2_segment(ps_ref, am_ref, local_ref, bias_ref, nov_ref, sal_ref,
                seg_ref, w_ref):
    lane = jax.lax.broadcasted_iota(jnp.int32, (1, L), 1)
    am = _t_row(am_ref)
    local = _t_row(local_ref)
    novelty = (am - local) * nov_ref[0, 0]
    sal = jax.nn.sigmoid(_t_row(ps_ref) + novelty + bias_ref[0, 0])
    sal_ref[0] = sal
    above = sal > THR
    abovef = above.astype(F32)
    prev = _shift_right(abovef, 1, lane)
    start = jnp.where(above, 1.0 - prev, 0.0)
    cum = start
    k = 1
    while k < L:
        cum = cum + _shift_right(cum, k, lane)
        k *= 2
    span = cum - 1.0
    seg_ref[0] = jnp.where(above, jnp.minimum(span, float(S)), float(S))
    w_ref[0] = jnp.where(above, sal, 0.0)


def _k3_pool(zr_ref, zi_ref, seg_ref, w_ref, evr_ref, evi_ref, em_ref,
             accr, acci, accw):
    li = pl.program_id(1)

    @pl.when(li == 0)
    def _():
        accr[...] = jnp.zeros_like(accr)
        acci[...] = jnp.zeros_like(acci)
        accw[...] = jnp.zeros_like(accw)

    seg = seg_ref[0]
    w = w_ref[0]
    sid = jax.lax.broadcasted_iota(jnp.int32, (S + 1, 1), 0).astype(F32)
    oht = (seg == sid).astype(F32)
    ohtw = oht * w
    accr[...] += jnp.dot(ohtw, zr_ref[0], preferred_element_type=F32,
                         precision=_HI)
    acci[...] += jnp.dot(ohtw, zi_ref[0], preferred_element_type=F32,
                         precision=_HI)
    den = jnp.sum(ohtw, axis=1, keepdims=True)
    cnt = jnp.sum(oht * (w > 0.0), axis=1, keepdims=True)
    accw[...] += jnp.concatenate([den, cnt], axis=1)

    @pl.when(li == NL - 1)
    def _():
        den_f = jnp.maximum(accw[:S, 0:1], 1e-8)
        evr_ref[0] = accr[:S, :] / den_f
        evi_ref[0] = acci[:S, :] / den_f
        em_ref[0] = (accw[:S, 1:2] > 0.0).astype(F32)


def _k4_project(evr_ref, evi_ref, em_ref, skr_ref, ski_ref, svr_ref, svi_ref,
                sm_ref, wkr_ref, wki_ref, wvr_ref, wvi_ref,
                nkr_ref, nki_ref, nvr_ref, nvi_ref, nm_ref):
    er = evr_ref[0]
    ei = evi_ref[0]
    m = em_ref[0]

    def clin(wr_ref, wi_ref):
        yr = (jnp.dot(er, wr_ref[...], preferred_element_type=F32)
              - jnp.dot(ei, wi_ref[...], preferred_element_type=F32))
        yi = (jnp.dot(er, wi_ref[...], preferred_element_type=F32)
              + jnp.dot(ei, wr_ref[...], preferred_element_type=F32))
        return yr, yi

    ekr, eki = clin(wkr_ref, wki_ref)
    evvr, evvi = clin(wvr_ref, wvi_ref)
    nkr_ref[0] = m * ekr + (1.0 - m) * skr_ref[0]
    nki_ref[0] = m * eki + (1.0 - m) * ski_ref[0]
    nvr_ref[0] = m * evvr + (1.0 - m) * svr_ref[0]
    nvi_ref[0] = m * evvi + (1.0 - m) * svi_ref[0]
    ones11 = jnp.ones((1, 1), F32)
    m_row = jax.lax.dot_general(ones11, m, (((1,), (1,)), ((), ())),
                                preferred_element_type=F32,
                                precision=_HI)
    nm_ref[0] = jnp.minimum(sm_ref[0] + m_row, 1.0)


def _k5_retrieve(zr_ref, zi_ref, rms_ref, g_ref, wqr_ref, wqi_ref,
                 nkr_ref, nki_ref, nvr_ref, nvi_ref, nm_ref,
                 or_ref, oi_ref):
    inv = g_ref[...] / rms_ref[0]
    znr = zr_ref[0] * inv
    zni = zi_ref[0] * inv
    qr = (jnp.dot(znr, wqr_ref[...], preferred_element_type=F32)
          - jnp.dot(zni, wqi_ref[...], preferred_element_type=F32))
    qi = (jnp.dot(znr, wqi_ref[...], preferred_element_type=F32)
          + jnp.dot(zni, wqr_ref[...], preferred_element_type=F32))
    dn = (((1,), (1,)), ((), ()))
    s = (jax.lax.dot_general(qr, nkr_ref[0], dn, preferred_element_type=F32)
         + jax.lax.dot_general(qi, nki_ref[0], dn, preferred_element_type=F32)
         ) * (1.0 / np.sqrt(DIM))
    s = s + (nm_ref[0] - 1.0) * 1e9
    cur = s
    m1 = jnp.max(cur, axis=1, keepdims=True)
    mk = m1
    for _ in range(TOPK - 1):
        cur = jnp.where(cur >= mk, -1e30, cur)
        mk = jnp.max(cur, axis=1, keepdims=True)
    w = jnp.where(s >= mk, jnp.exp(s - m1), 0.0)
    w = w / jnp.sum(w, axis=1, keepdims=True)
    or_ref[0] = jnp.dot(w, nvr_ref[0], preferred_element_type=F32)
    oi_ref[0] = jnp.dot(w, nvi_ref[0], preferred_element_type=F32)


@jax.jit
def kernel(z, slot_keys, slot_values, slot_mask, sal_Wr, sal_Wi, score_bias,
           novelty_scale, key_Wr, key_Wi, val_Wr, val_Wi, qry_Wr, qry_Wi,
           norm_gamma):
    zr, zi = z[..., 0], z[..., 1]
    skr, ski = slot_keys[..., 0], slot_keys[..., 1]
    svr, svi = slot_values[..., 0], slot_values[..., 1]
    sm3 = slot_mask.reshape(B, 1, S)
    g = norm_gamma.reshape(1, DIM)
    bias = score_bias.reshape(1, 1).astype(F32)
    nov = novelty_scale.reshape(1, 1).astype(F32)

    zspec = pl.BlockSpec((1, LB, DIM), lambda b, l: (b, l, 0))
    colspec = pl.BlockSpec((1, LB, 1), lambda b, l: (b, l, 0))
    rowspec = pl.BlockSpec((1, 1, LB), lambda b, l: (b, 0, l))
    vecspec = pl.BlockSpec((DIM, 1), lambda b, l: (0, 0))

    ps, am, rms = pl.pallas_call(
        _k1_stats,
        grid=(B, NL),
        in_specs=[zspec, zspec, vecspec, vecspec],
        out_specs=[colspec, colspec, colspec],
        out_shape=[jax.ShapeDtypeStruct((B, L, 1), F32)] * 3,
    )(zr, zi, sal_Wr, sal_Wi)

    fullrow = pl.BlockSpec((1, 1, L), lambda b: (b, 0, 0))
    fullcol = pl.BlockSpec((1, L, 1), lambda b: (b, 0, 0))
    scalar = pl.BlockSpec((1, 1), lambda b: (0, 0))
    kern5 = jnp.ones((5,), F32) / 5.0
    local = jax.vmap(lambda a: jnp.convolve(a, kern5, mode='same'))(
        am.reshape(B, L)).reshape(B, L, 1)

    sal3, seg, wgt = pl.pallas_call(
        _k2_segment,
        grid=(B,),
        in_specs=[fullcol, fullcol, fullcol, scalar, scalar],
        out_specs=[fullrow, fullrow, fullrow],
        out_shape=[jax.ShapeDtypeStruct((B, 1, L), F32)] * 3,
    )(ps, am, local, bias, nov)

    evspec = pl.BlockSpec((1, S, DIM), lambda b, l: (b, 0, 0))
    evr, evi, emask = pl.pallas_call(
        _k3_pool,
        grid=(B, NL),
        in_specs=[zspec, zspec, rowspec, rowspec],
        out_specs=[evspec, evspec,
                   pl.BlockSpec((1, S, 1), lambda b, l: (b, 0, 0))],
        out_shape=[jax.ShapeDtypeStruct((B, S, DIM), F32),
                   jax.ShapeDtypeStruct((B, S, DIM), F32),
                   jax.ShapeDtypeStruct((B, S, 1), F32)],
        scratch_shapes=[pltpu.VMEM((S + 1, DIM), F32),
                        pltpu.VMEM((S + 1, DIM), F32),
                        pltpu.VMEM((S + 1, 2), F32)],
    )(zr, zi, seg, wgt)

    slotspec = pl.BlockSpec((1, S, DIM), lambda b: (b, 0, 0))
    wspec = pl.BlockSpec((DIM, DIM), lambda b: (0, 0))
    maskspec = pl.BlockSpec((1, 1, S), lambda b: (b, 0, 0))
    nkr, nki, nvr, nvi, nm3 = pl.pallas_call(
        _k4_project,
        grid=(B,),
        in_specs=[slotspec, slotspec,
                  pl.BlockSpec((1, S, 1), lambda b: (b, 0, 0)),
                  slotspec, slotspec, slotspec, slotspec, maskspec,
                  wspec, wspec, wspec, wspec],
        out_specs=[slotspec, slotspec, slotspec, slotspec, maskspec],
        out_shape=[jax.ShapeDtypeStruct((B, S, DIM), F32)] * 4
        + [jax.ShapeDtypeStruct((B, 1, S), F32)],
    )(evr, evi, emask, skr, ski, svr, svi, sm3, key_Wr, key_Wi,
      val_Wr, val_Wi)

    retr, reti = pl.pallas_call(
        _k5_retrieve,
        grid=(B, NL),
        in_specs=[zspec, zspec, colspec,
                  pl.BlockSpec((1, DIM), lambda b, l: (0, 0)),
                  pl.BlockSpec((DIM, DIM), lambda b, l: (0, 0)),
                  pl.BlockSpec((DIM, DIM), lambda b, l: (0, 0)),
                  pl.BlockSpec((1, S, DIM), lambda b, l: (b, 0, 0)),
                  pl.BlockSpec((1, S, DIM), lambda b, l: (b, 0, 0)),
                  pl.BlockSpec((1, S, DIM), lambda b, l: (b, 0, 0)),
                  pl.BlockSpec((1, S, DIM), lambda b, l: (b, 0, 0)),
                  pl.BlockSpec((1, 1, S), lambda b, l: (b, 0, 0))],
        out_specs=[zspec, zspec],
        out_shape=[jax.ShapeDtypeStruct((B, L, DIM), F32)] * 2,
    )(zr, zi, rms, g, qry_Wr, qry_Wi, nkr, nki, nvr, nvi, nm3)

    retrieved = jnp.stack([retr, reti], axis=-1)
    new_keys = jnp.stack([nkr, nki], axis=-1)
    new_values = jnp.stack([nvr, nvi], axis=-1)
    return retrieved, new_keys, new_values, nm3.reshape(B, S), sal3.reshape(B, L)
